# baseline (speedup 1.0000x reference)
import jax
import jax.numpy as jnp
from jax.experimental import pallas as pl
from jax.experimental.pallas import tpu as pltpu


def _moe_block_kernel(x_ref, w1f_ref, w2f_ref, w1a_ref, w1s_ref,
                      rowbias_ref, wr2_ref, br2_ref, wr3_ref, br3_ref,
                      be1_ref, be2_ref, out_ref, we1_ref, we2_ref,
                      stage_ref, sems):
    step = pl.program_id(0)
    num_experts = we1_ref.shape[0]

    def copy(k):
        e = k // 2
        src = w1f_ref.at[e] if k % 2 == 0 else w2f_ref.at[e]
        return pltpu.make_async_copy(src, stage_ref.at[k % 2],
                                     sems.at[k % 2])

    def body(land_weights):
        xb = x_ref[...]
        xbf = xb.astype(jnp.bfloat16)

        m = jnp.mean(xb, axis=1, keepdims=True)
        sd = jnp.sqrt(jnp.mean((xb - m) ** 2, axis=1, keepdims=True))
        mx = jnp.max(xb, axis=1, keepdims=True)
        mn = jnp.min(xb, axis=1, keepdims=True)

        h = jnp.dot(xbf, w1a_ref[0], preferred_element_type=jnp.float32)
        h += rowbias_ref[...]
        h += m * w1s_ref[0:1, :]
        h += sd * w1s_ref[1:2, :]
        h += mx * w1s_ref[2:3, :]
        h += mn * w1s_ref[3:4, :]
        h = jnp.maximum(h, 0.0).astype(jnp.bfloat16)

        h2 = jnp.dot(h, wr2_ref[0], preferred_element_type=jnp.float32)
        h2 += br2_ref[...]
        h2 = jnp.maximum(h2, 0.0).astype(jnp.bfloat16)

        scores = jnp.dot(h2, wr3_ref[0], preferred_element_type=jnp.float32)
        scores += br3_ref[...]
        scores -= jnp.max(scores, axis=1, keepdims=True)
        eg = jnp.exp(scores)
        gates = eg / jnp.sum(eg, axis=1, keepdims=True)

        for e in range(num_experts):
            if land_weights:
                copy(2 * e).wait()
                we1_ref[e] = stage_ref[0].astype(jnp.bfloat16)
                if 2 * e + 2 < 2 * num_experts:
                    copy(2 * e + 2).start()
                copy(2 * e + 1).wait()
                we2_ref[e] = stage_ref[1].astype(jnp.bfloat16)
                if 2 * e + 3 < 2 * num_experts:
                    copy(2 * e + 3).start()
            h1 = jnp.dot(xbf, we1_ref[e], preferred_element_type=jnp.float32)
            h1 += be1_ref[e:e + 1, :]
            h1 = jnp.maximum(h1, 0.0)
            g = gates[:, e:e + 1]
            h1g = (h1 * g).astype(jnp.bfloat16)
            part = jnp.dot(h1g, we2_ref[e],
                           preferred_element_type=jnp.float32)
            part += g * be2_ref[e:e + 1, :]
            if e == 0:
                out_ref[...] = part
            else:
                out_ref[...] += part

    @pl.when(step == 0)
    def _first_step():
        copy(0).start()
        copy(1).start()
        body(True)

    @pl.when(step != 0)
    def _steady_state():
        body(False)


def kernel(x, Wr1, br1, Wr2, br2, Wr3, br3, temperature, W1, b1, W2, b2):
    n, d = x.shape
    e = W1.shape[0]
    dh = Wr2.shape[1]
    bn = min(512, n)

    gmean = jnp.mean(x, axis=0, keepdims=True)
    rowbias = gmean @ Wr1[d:2 * d, :] + br1[None, :]
    w1a = Wr1[:d, :].astype(jnp.bfloat16)
    w1s = Wr1[2 * d:, :]
    wr2 = Wr2.astype(jnp.bfloat16)
    inv_t = 1.0 / temperature[0]
    wr3 = (Wr3 * inv_t).astype(jnp.bfloat16)
    br3s = (br3 * inv_t)[None, :]

    const2 = lambda i: (0, 0)
    const3 = lambda i: (0, 0, 0)
    return pl.pallas_call(
        _moe_block_kernel,
        grid=(n // bn,),
        in_specs=[
            pl.BlockSpec((bn, d), lambda i: (i, 0)),
            pl.BlockSpec(memory_space=pl.ANY),
            pl.BlockSpec(memory_space=pl.ANY),
            pl.BlockSpec((1, d, d), const3),
            pl.BlockSpec((4, d), const2),
            pl.BlockSpec((1, d), const2),
            pl.BlockSpec((1, d, dh), const3),
            pl.BlockSpec((1, dh), const2),
            pl.BlockSpec((1, dh, e), const3),
            pl.BlockSpec((1, e), const2),
            pl.BlockSpec((e, d), const2),
            pl.BlockSpec((e, d), const2),
        ],
        out_specs=pl.BlockSpec((bn, d), lambda i: (i, 0)),
        out_shape=jax.ShapeDtypeStruct((n, d), jnp.float32),
        scratch_shapes=[
            pltpu.VMEM((e, d, d), jnp.bfloat16),
            pltpu.VMEM((e, d, d), jnp.bfloat16),
            pltpu.VMEM((2, d, d), jnp.float32),
            pltpu.SemaphoreType.DMA((2,)),
        ],
        compiler_params=pltpu.CompilerParams(
            dimension_semantics=("arbitrary",)),
    )(x, W1, W2, w1a[None], w1s, rowbias, wr2[None], br2[None, :],
      wr3[None], br3s, b1, b2)


# speedup vs baseline: 1.0422x; 1.0422x over previous
import jax
import jax.numpy as jnp
from jax.experimental import pallas as pl
from jax.experimental.pallas import tpu as pltpu


def _moe_block_kernel(x_ref, w1f_ref, w2f_ref, w1a_ref, w1s_ref,
                      rowbias_ref, wr2_ref, br2_ref, wr3_ref, br3_ref,
                      be1_ref, be2_ref, out_ref, we1_ref, we2_ref,
                      stage_ref, sems):
    step = pl.program_id(0)
    num_experts = we1_ref.shape[0]

    def copy(k):
        e = k // 2
        src = w1f_ref.at[e] if k % 2 == 0 else w2f_ref.at[e]
        return pltpu.make_async_copy(src, stage_ref.at[k % 2],
                                     sems.at[k % 2])

    @pl.when(step == 0)
    def _start_first_copies():
        copy(0).start()
        copy(1).start()

    xb = x_ref[...]
    xbf = xb.astype(jnp.bfloat16)

    m = jnp.mean(xb, axis=1, keepdims=True)
    sd = jnp.sqrt(jnp.mean((xb - m) ** 2, axis=1, keepdims=True))
    mx = jnp.max(xb, axis=1, keepdims=True)
    mn = jnp.min(xb, axis=1, keepdims=True)

    h = jnp.dot(xbf, w1a_ref[0], preferred_element_type=jnp.float32)
    h += rowbias_ref[...]
    h += m * w1s_ref[0:1, :]
    h += sd * w1s_ref[1:2, :]
    h += mx * w1s_ref[2:3, :]
    h += mn * w1s_ref[3:4, :]
    h = jnp.maximum(h, 0.0).astype(jnp.bfloat16)

    h2 = jnp.dot(h, wr2_ref[0], preferred_element_type=jnp.float32)
    h2 += br2_ref[...]
    h2 = jnp.maximum(h2, 0.0).astype(jnp.bfloat16)

    scores = jnp.dot(h2, wr3_ref[0], preferred_element_type=jnp.float32)
    scores += br3_ref[...]
    scores -= jnp.max(scores, axis=1, keepdims=True)
    eg = jnp.exp(scores)
    gates = eg / jnp.sum(eg, axis=1, keepdims=True)

    @pl.when(step == 0)
    def _land_weights():
        for k in range(2 * num_experts):
            copy(k).wait()
            val = stage_ref[k % 2].astype(jnp.bfloat16)
            if k % 2 == 0:
                we1_ref[k // 2] = val
            else:
                we2_ref[k // 2] = val
            if k + 2 < 2 * num_experts:
                copy(k + 2).start()

    for e in range(num_experts):
        h1 = jnp.dot(xbf, we1_ref[e], preferred_element_type=jnp.float32)
        h1 += be1_ref[e:e + 1, :]
        h1 = jnp.maximum(h1, 0.0)
        g = gates[:, e:e + 1]
        h1g = (h1 * g).astype(jnp.bfloat16)
        part = jnp.dot(h1g, we2_ref[e], preferred_element_type=jnp.float32)
        part += g * be2_ref[e:e + 1, :]
        if e == 0:
            out_ref[...] = part
        else:
            out_ref[...] += part


def kernel(x, Wr1, br1, Wr2, br2, Wr3, br3, temperature, W1, b1, W2, b2):
    n, d = x.shape
    e = W1.shape[0]
    dh = Wr2.shape[1]
    bn = min(512, n)

    gmean = jnp.mean(x, axis=0, keepdims=True)
    rowbias = gmean @ Wr1[d:2 * d, :] + br1[None, :]
    w1a = Wr1[:d, :].astype(jnp.bfloat16)
    w1s = Wr1[2 * d:, :]
    wr2 = Wr2.astype(jnp.bfloat16)
    inv_t = 1.0 / temperature[0]
    wr3 = (Wr3 * inv_t).astype(jnp.bfloat16)
    br3s = (br3 * inv_t)[None, :]

    const2 = lambda i: (0, 0)
    const3 = lambda i: (0, 0, 0)
    return pl.pallas_call(
        _moe_block_kernel,
        grid=(n // bn,),
        in_specs=[
            pl.BlockSpec((bn, d), lambda i: (i, 0)),
            pl.BlockSpec(memory_space=pl.ANY),
            pl.BlockSpec(memory_space=pl.ANY),
            pl.BlockSpec((1, d, d), const3),
            pl.BlockSpec((4, d), const2),
            pl.BlockSpec((1, d), const2),
            pl.BlockSpec((1, d, dh), const3),
            pl.BlockSpec((1, dh), const2),
            pl.BlockSpec((1, dh, e), const3),
            pl.BlockSpec((1, e), const2),
            pl.BlockSpec((e, d), const2),
            pl.BlockSpec((e, d), const2),
        ],
        out_specs=pl.BlockSpec((bn, d), lambda i: (i, 0)),
        out_shape=jax.ShapeDtypeStruct((n, d), jnp.float32),
        scratch_shapes=[
            pltpu.VMEM((e, d, d), jnp.bfloat16),
            pltpu.VMEM((e, d, d), jnp.bfloat16),
            pltpu.VMEM((2, d, d), jnp.float32),
            pltpu.SemaphoreType.DMA((2,)),
        ],
        compiler_params=pltpu.CompilerParams(
            dimension_semantics=("arbitrary",)),
    )(x, W1, W2, w1a[None], w1s, rowbias, wr2[None], br2[None, :],
      wr3[None], br3s, b1, b2)


# speedup vs baseline: 1.0520x; 1.0094x over previous
import jax
import jax.numpy as jnp
from jax.experimental import pallas as pl
from jax.experimental.pallas import tpu as pltpu


def _moe_block_kernel(x_ref, w1f_ref, w2f_ref, w1a_ref, w1s_ref,
                      rowbias_ref, wr2_ref, br2_ref, wr3_ref, br3_ref,
                      be1_ref, be2_ref, out_ref, we1_ref, we2_ref,
                      stage_ref, sems):
    step = pl.program_id(0)
    num_experts = we1_ref.shape[0]

    @pl.when(step == 0)
    def _prologue():
        def copy(k):
            e = k // 2
            src = w1f_ref.at[e] if k % 2 == 0 else w2f_ref.at[e]
            return pltpu.make_async_copy(src, stage_ref.at[k % 2],
                                         sems.at[k % 2])

        copy(0).start()
        for k in range(2 * num_experts):
            if k + 1 < 2 * num_experts:
                copy(k + 1).start()
            copy(k).wait()
            val = stage_ref[k % 2].astype(jnp.bfloat16)
            if k % 2 == 0:
                we1_ref[k // 2] = val
            else:
                we2_ref[k // 2] = val

    xb = x_ref[...]
    xbf = xb.astype(jnp.bfloat16)

    m = jnp.mean(xb, axis=1, keepdims=True)
    sd = jnp.sqrt(jnp.mean((xb - m) ** 2, axis=1, keepdims=True))
    mx = jnp.max(xb, axis=1, keepdims=True)
    mn = jnp.min(xb, axis=1, keepdims=True)

    h = jnp.dot(xbf, w1a_ref[0], preferred_element_type=jnp.float32)
    h += rowbias_ref[...]
    h += m * w1s_ref[0:1, :]
    h += sd * w1s_ref[1:2, :]
    h += mx * w1s_ref[2:3, :]
    h += mn * w1s_ref[3:4, :]
    h = jnp.maximum(h, 0.0).astype(jnp.bfloat16)

    h2 = jnp.dot(h, wr2_ref[0], preferred_element_type=jnp.float32)
    h2 += br2_ref[...]
    h2 = jnp.maximum(h2, 0.0).astype(jnp.bfloat16)

    scores = jnp.dot(h2, wr3_ref[0], preferred_element_type=jnp.float32)
    scores += br3_ref[...]
    scores -= jnp.max(scores, axis=1, keepdims=True)
    eg = jnp.exp(scores)
    gates = eg / jnp.sum(eg, axis=1, keepdims=True)

    for e in range(num_experts):
        h1 = jnp.dot(xbf, we1_ref[e], preferred_element_type=jnp.float32)
        h1 += be1_ref[e:e + 1, :]
        h1 = jnp.maximum(h1, 0.0)
        g = gates[:, e:e + 1]
        h1g = (h1 * g).astype(jnp.bfloat16)
        part = jnp.dot(h1g, we2_ref[e], preferred_element_type=jnp.float32)
        part += g * be2_ref[e:e + 1, :]
        if e == 0:
            out_ref[...] = part
        else:
            out_ref[...] += part


def kernel(x, Wr1, br1, Wr2, br2, Wr3, br3, temperature, W1, b1, W2, b2):
    n, d = x.shape
    e = W1.shape[0]
    dh = Wr2.shape[1]
    bn = min(512, n)

    gmean = jnp.mean(x, axis=0, keepdims=True)
    rowbias = gmean @ Wr1[d:2 * d, :] + br1[None, :]
    w1a = Wr1[:d, :].astype(jnp.bfloat16)
    w1s = Wr1[2 * d:, :]
    wr2 = Wr2.astype(jnp.bfloat16)
    inv_t = 1.0 / temperature[0]
    wr3 = (Wr3 * inv_t).astype(jnp.bfloat16)
    br3s = (br3 * inv_t)[None, :]

    const2 = lambda i: (0, 0)
    const3 = lambda i: (0, 0, 0)
    return pl.pallas_call(
        _moe_block_kernel,
        grid=(n // bn,),
        in_specs=[
            pl.BlockSpec((bn, d), lambda i: (i, 0)),
            pl.BlockSpec(memory_space=pl.ANY),
            pl.BlockSpec(memory_space=pl.ANY),
            pl.BlockSpec((1, d, d), const3),
            pl.BlockSpec((4, d), const2),
            pl.BlockSpec((1, d), const2),
            pl.BlockSpec((1, d, dh), const3),
            pl.BlockSpec((1, dh), const2),
            pl.BlockSpec((1, dh, e), const3),
            pl.BlockSpec((1, e), const2),
            pl.BlockSpec((e, d), const2),
            pl.BlockSpec((e, d), const2),
        ],
        out_specs=pl.BlockSpec((bn, d), lambda i: (i, 0)),
        out_shape=jax.ShapeDtypeStruct((n, d), jnp.float32),
        scratch_shapes=[
            pltpu.VMEM((e, d, d), jnp.bfloat16),
            pltpu.VMEM((e, d, d), jnp.bfloat16),
            pltpu.VMEM((2, d, d), jnp.float32),
            pltpu.SemaphoreType.DMA((2,)),
        ],
        compiler_params=pltpu.CompilerParams(
            dimension_semantics=("arbitrary",)),
    )(x, W1, W2, w1a[None], w1s, rowbias, wr2[None], br2[None, :],
      wr3[None], br3s, b1, b2)
